# Initial kernel scaffold; baseline (speedup 1.0000x reference)
import jax
import jax.numpy as jnp
from jax.experimental import pallas as pl


def kernel(dr_vec, Z, neighbor_idxs, embeddings):
    raise NotImplementedError("write your pallas kernel here")


# speedup vs baseline: 199.9388x; 199.9388x over previous
import functools

import numpy as np
import jax
import jax.numpy as jnp
from jax import lax
from jax.experimental import pallas as pl
from jax.experimental.pallas import tpu as pltpu
from jax.experimental.pallas import tpu_sc as plsc

N_ATOMS = 10000
N_EDGES = 320000
N_SPECIES = 119
NR = 5
NB = 7
R_MIN, R_MAX = 0.5, 6.0

ROW = 112
NPAIR = N_SPECIES * N_SPECIES
EMB_W = 48
NC, NS, L = 2, 16, 16
NW = NC * NS
EPW = N_EDGES // NW
CH = 80
NCH = EPW // CH
GPC = CH // L
NPAD = 10240
APT = NPAD // NS

U2 = [(0, 0), (0, 1), (0, 2), (1, 1), (1, 2), (2, 2)]
U3 = [(0, 0, 0), (0, 0, 1), (0, 0, 2), (0, 1, 1), (0, 1, 2), (0, 2, 2),
      (1, 1, 1), (1, 1, 2), (1, 2, 2), (2, 2, 2)]
U2_OF = {}
for _u, (_i, _j) in enumerate(U2):
    U2_OF[(_i, _j)] = _u
    U2_OF[(_j, _i)] = _u
U3_OF = {}
for _u, (_i, _j, _k) in enumerate(U3):
    import itertools as _it
    for _p in set(_it.permutations((_i, _j, _k))):
        U3_OF[_p] = _u
W2 = [1.0, 2.0, 2.0, 1.0, 2.0, 1.0]
W3 = [1.0, 3.0, 3.0, 3.0, 6.0, 3.0, 1.0, 3.0, 3.0, 1.0]
SYMIJ = [(i, j, (1.0 if i == j else 2.0)) for i in range(3) for j in range(i, 3)]
T2 = [(r, s) for r in range(NR) for s in range(r, NR)]
T3 = [(r, s, t) for r in range(NR) for s in range(r, NR) for t in range(s, NR)]

BETTA = NB * NB / (R_MAX * R_MAX)
RAD_NORM = float((2.0 * BETTA / np.pi) ** 0.25)
EMB_NORM = float(1.0 / np.sqrt(NB))
SCALE = RAD_NORM * EMB_NORM
SHIFTS = [float(R_MIN + (R_MAX - R_MIN) / NB * b) for b in range(NB)]
COS_C = [0.9999999999193523, -4.934802189552149, 4.058711882088068,
         -1.3352607090906639, 0.23532212776844633, -0.02578785261756146,
         0.0019059102695288554, -8.916918755985427e-05]


def _sc_body(drx_h, dry_h, drz_h, ii_h, ij_h, z_h, emb_h, out_h,
             zv, dxv, dyv, dzv, iiv, ijv, pairv, embv, outv, accs, sem):
    cid = lax.axis_index("c")
    sid = lax.axis_index("s")
    wid = sid * NC + cid
    base = wid * EPW
    zero16 = jnp.zeros((L,), jnp.float32)
    iota = lax.iota(jnp.int32, L)

    def zrow(r, c):
        for k in range(ROW // L):
            outv[r, pl.ds(k * L, L)] = zero16
        return c
    lax.fori_loop(0, CH, zrow, 0)

    a0 = sid * APT
    nfull = APT // CH
    for k in range(nfull):
        pltpu.sync_copy(outv, accs.at[pl.ds(a0 + k * CH, CH)])

    pltpu.sync_copy(z_h, zv)
    plsc.subcore_barrier()

    def chunk(c0, carry):
        eb = base + c0 * CH
        pltpu.sync_copy(ii_h.at[pl.ds(eb, CH)], iiv)
        pltpu.sync_copy(ij_h.at[pl.ds(eb, CH)], ijv)
        pltpu.sync_copy(drx_h.at[pl.ds(eb, CH)], dxv)
        pltpu.sync_copy(dry_h.at[pl.ds(eb, CH)], dyv)
        pltpu.sync_copy(drz_h.at[pl.ds(eb, CH)], dzv)

        def mkpair(g, c):
            off = g * L
            zi = plsc.load_gather(zv, [iiv[pl.ds(off, L)]])
            zj = plsc.load_gather(zv, [ijv[pl.ds(off, L)]])
            pairv[pl.ds(off, L)] = zi * N_SPECIES + zj
            return c
        lax.fori_loop(0, GPC, mkpair, 0)

        pltpu.async_copy(emb_h.at[pairv], embv, sem).wait()

        def group(g, c):
            off = g * L
            rows = off + iota
            ii = iiv[pl.ds(off, L)]
            ij = ijv[pl.ds(off, L)]
            dx = dxv[pl.ds(off, L)]
            dy = dyv[pl.ds(off, L)]
            dz = dzv[pl.ds(off, L)]
            s = dx * dx + dy * dy + dz * dz
            s_c = jnp.maximum(s, 1e-30)
            y = plsc.bitcast(
                jnp.int32(0x5F3759DF) - (plsc.bitcast(s_c, jnp.int32) >> 1),
                jnp.float32)
            for _ in range(3):
                y = y * (1.5 - 0.5 * s_c * y * y)
            dr = s * y
            inv = 1.0 / (dr + 1e-5)
            nx, ny, nz = dx * inv, dy * inv, dz * inv
            u = dr * dr * (1.0 / (R_MAX * R_MAX))
            cv = jnp.full((L,), COS_C[-1], jnp.float32)
            for k in range(len(COS_C) - 2, -1, -1):
                cv = cv * u + COS_C[k]
            cut = jnp.where(dr < R_MAX, 0.5 * (cv + 1.0), 0.0)
            f = jnp.where(ii != ij, cut * SCALE, 0.0)
            es = []
            for b in range(NB):
                t = dr - SHIFTS[b]
                es.append(jnp.exp((-BETTA) * (t * t)))
            rad = []
            for r in range(NR):
                a = None
                for b in range(NB):
                    cc = plsc.load_gather(
                        embv, [rows, jnp.full((L,), r * NB + b, jnp.int32)])
                    a = cc * es[b] if a is None else a + cc * es[b]
                rad.append(a * f)
            mono1 = [nx, ny, nz]
            m2v = [mono1[i] * mono1[j] for (i, j) in U2]
            m3v = [m2v[U2_OF[(i, j)]] * mono1[k] for (i, j, k) in U3]

            def put(col, val):
                plsc.store_scatter(
                    outv, [rows, jnp.full((L,), col, jnp.int32)], val)
            for r in range(NR):
                put(r, rad[r])
                for i in range(3):
                    put(5 + r * 3 + i, rad[r] * mono1[i])
                for uu in range(6):
                    put(20 + r * 6 + uu, rad[r] * m2v[uu])
                for uu in range(10):
                    put(50 + r * 10 + uu, rad[r] * m3v[uu])
            return c
        lax.fori_loop(0, GPC, group, 0)

        pltpu.sync_copy(outv, accs.at[ijv], add=True)
        return carry
    lax.fori_loop(0, NCH, chunk, 0)

    plsc.subcore_barrier()

    for k in range(nfull):
        pltpu.sync_copy(accs.at[pl.ds(a0 + k * CH, CH)], outv)
        pltpu.sync_copy(outv, out_h.at[cid, pl.ds(a0 + k * CH, CH)])


@functools.lru_cache(maxsize=1)
def _build_sc():
    return pl.kernel(
        _sc_body,
        out_type=jax.ShapeDtypeStruct((NC, NPAD, ROW), jnp.float32),
        mesh=plsc.VectorSubcoreMesh(core_axis_name="c", subcore_axis_name="s"),
        compiler_params=pltpu.CompilerParams(
            needs_layout_passes=False, use_tc_tiling_on_sc=False),
        scratch_types=[
            pltpu.VMEM((N_ATOMS,), jnp.int32),
            pltpu.VMEM((CH,), jnp.float32),
            pltpu.VMEM((CH,), jnp.float32),
            pltpu.VMEM((CH,), jnp.float32),
            pltpu.VMEM((CH,), jnp.int32),
            pltpu.VMEM((CH,), jnp.int32),
            pltpu.VMEM((CH,), jnp.int32),
            pltpu.VMEM((CH, EMB_W), jnp.float32),
            pltpu.VMEM((CH, ROW), jnp.float32),
            pltpu.VMEM_SHARED((NPAD, ROW), jnp.float32),
            pltpu.SemaphoreType.DMA,
        ],
    )


def _tc_body(x_ref, o_ref):
    def comp(m):
        return x_ref[0, m] + x_ref[1, m]

    m0 = [comp(r) for r in range(NR)]
    m1 = [[comp(5 + r * 3 + i) for i in range(3)] for r in range(NR)]
    m2u = [[comp(20 + r * 6 + u) for u in range(6)] for r in range(NR)]
    m3u = [[comp(50 + r * 10 + u) for u in range(10)] for r in range(NR)]

    def m2(r, i, j):
        return m2u[r][U2_OF[(i, j)]]

    def m3(r, i, j, k):
        return m3u[r][U3_OF[(i, j, k)]]

    n = 0
    for r in range(NR):
        o_ref[n] = m0[r]
        n += 1
    for (r, s) in T2:
        o_ref[n] = (m1[r][0] * m1[s][0] + m1[r][1] * m1[s][1]
                    + m1[r][2] * m1[s][2])
        n += 1
    for (r, s) in T2:
        v = None
        for u in range(6):
            t = (W2[u] * m2u[r][u]) * m2u[s][u]
            v = t if v is None else v + t
        o_ref[n] = v
        n += 1
    for (r, s) in T2:
        v = None
        for u in range(10):
            t = (W3[u] * m3u[r][u]) * m3u[s][u]
            v = t if v is None else v + t
        o_ref[n] = v
        n += 1
    for r in range(NR):
        for s in range(r, NR):
            B = [[None] * 3 for _ in range(3)]
            for j in range(3):
                for k in range(3):
                    v = None
                    for i in range(3):
                        t = m2(r, i, j) * m2(s, i, k)
                        v = t if v is None else v + t
                    B[j][k] = v
            for t in range(s, NR):
                v = None
                for j in range(3):
                    for k in range(3):
                        tt = B[j][k] * m2(t, j, k)
                        v = tt if v is None else v + tt
                o_ref[n] = v
                n += 1
    v5 = [[[None] * 3 for _ in range(NR)] for _ in range(NR)]
    for s in range(NR):
        for t in range(NR):
            for i in range(3):
                v = None
                for j in range(3):
                    tt = m1[s][j] * m2(t, i, j)
                    v = tt if v is None else v + tt
                v5[s][t][i] = v
    for (r, s) in T2:
        for t in range(NR):
            o_ref[n] = (m1[r][0] * v5[s][t][0] + m1[r][1] * v5[s][t][1]
                        + m1[r][2] * v5[s][t][2])
            n += 1
    for (r, s) in T2:
        A = [[None] * 3 for _ in range(3)]
        for k in range(3):
            for ll in range(3):
                v = None
                for (i, j, w) in SYMIJ:
                    tt = (w * m3(r, i, j, k)) * m3(s, i, j, ll)
                    v = tt if v is None else v + tt
                A[k][ll] = v
        for t in range(NR):
            v = None
            for k in range(3):
                for ll in range(3):
                    tt = A[k][ll] * m2(t, k, ll)
                    v = tt if v is None else v + tt
            o_ref[n] = v
            n += 1
    for r in range(NR):
        for s in range(NR):
            d = [None] * 3
            for k in range(3):
                v = None
                for (i, j, w) in SYMIJ:
                    tt = (w * m3(r, i, j, k)) * m2(s, i, j)
                    v = tt if v is None else v + tt
                d[k] = v
            for t in range(NR):
                o_ref[n] = (d[0] * m1[t][0] + d[1] * m1[t][1]
                            + d[2] * m1[t][2])
                n += 1
    assert n == 360


_tc_contract = pl.pallas_call(
    _tc_body,
    grid=(NPAD // 1024,),
    in_specs=[pl.BlockSpec((NC, ROW, 8, 128), lambda i: (0, 0, i, 0))],
    out_specs=pl.BlockSpec((360, 8, 128), lambda i: (0, i, 0)),
    out_shape=jax.ShapeDtypeStruct((360, NPAD // 128, 128), jnp.float32),
)


def kernel(dr_vec, Z, neighbor_idxs, embeddings):
    drT = dr_vec.astype(jnp.float32).T
    idx_i = neighbor_idxs[0].astype(jnp.int32)
    idx_j = neighbor_idxs[1].astype(jnp.int32)
    emb = embeddings.astype(jnp.float32).reshape(NPAIR, NR * NB)
    emb = jnp.pad(emb, ((0, 0), (0, EMB_W - NR * NB)))
    z32 = Z.astype(jnp.int32)

    acc = _build_sc()(drT[0], drT[1], drT[2], idx_i, idx_j, z32, emb)
    accT = jnp.transpose(acc, (0, 2, 1))
    accT = accT.reshape(NC, ROW, NPAD // 128, 128)
    out = _tc_contract(accT)
    return out.reshape(360, NPAD)[:, :N_ATOMS].T
